# baseline (speedup 1.0000x reference)
import functools

import jax
import jax.numpy as jnp
from jax import lax
from jax.experimental import pallas as pl
from jax.experimental.pallas import tpu as pltpu
from jax.experimental.pallas import tpu_sc as plsc

N = 1048576
B = 131072
T = 0.5

NC = 2
NS = 16
NW = NC * NS
R = N // NW
IC = 8192
NCH = B // IC
C = 4096
NQ = R // C
L = 16
G = 512


def _dg(a, i):
    dnums = lax.GatherDimensionNumbers(
        offset_dims=(), collapsed_slice_dims=(0,), start_index_map=(0,))
    return lax.gather(a, i[:, None], dnums, (1,),
                      mode=lax.GatherScatterMode.PROMISE_IN_BOUNDS)


def _body(mp, mv, vp, vv, ix, out, w, idx0, civ, cjv, pos0, vel0, gj, gp, gi,
          gpx, gvx, gsem):
    wid = lax.axis_index("s") * NC + lax.axis_index("c")
    lo = wid * R
    iota = lax.iota(jnp.int32, L)
    lane15 = jnp.full((L,), L - 1, jnp.int32)

    def to_scalar(splat):
        return splat[L - 1]

    neg1 = jnp.full((L,), -1, jnp.int32)

    def init_w(i, _):
        for u in range(4):
            w[pl.ds(i * 4 * L + u * L, L)] = neg1
        return 0

    lax.fori_loop(0, R // (4 * L), init_w, 0)

    def scan_chunk(c, _):
        pltpu.sync_copy(ix.at[pl.ds(c * IC, IC)], idx0)

        def scan_vreg(k, cnt_splat):
            iv = idx0[pl.ds(k * L, L)]
            jv = c * IC + k * L + iota
            m = (iv >= lo) & (iv < lo + R)
            pc = plsc.all_reduce_population_count(m)

            @pl.when(pc[0] != 0)
            def _store():
                cum = plsc.cumsum(m.astype(jnp.int32))
                pos = cnt_splat + cum - 1
                plsc.store_scatter(civ, [pos], iv - lo, mask=m)
                plsc.store_scatter(cjv, [pos], jv, mask=m)

            return cnt_splat + pc

        cnt_splat = lax.fori_loop(0, IC // L, scan_vreg,
                                  jnp.zeros((L,), jnp.int32))
        cnt = to_scalar(cnt_splat)

        def apply_vreg(g, _):
            base_l = g * L
            av = civ[pl.ds(base_l, L)] & (R - 1)
            bv = cjv[pl.ds(base_l, L)]
            rem = cnt - base_l
            lm = iota < rem
            loser = jnp.zeros((L,), jnp.bool_)
            for r in range(1, L):
                rot = _dg(av, (iota + r) & (L - 1))
                eq = rot == av
                ok = (iota < (L - r)) & (iota < (rem - r))
                loser = loser | (eq & ok)
            win = lm & jnp.logical_not(loser)
            plsc.store_scatter(w, [av], bv, mask=win)
            return 0

        ng = (cnt + (L - 1)) >> 4
        lax.fori_loop(0, ng, apply_vreg, 0)
        return 0

    lax.fori_loop(0, NCH, scan_chunk, 0)

    def prefill(k, _):
        gi[pl.ds(k * L, L)] = (wid * 2 * C + k * L + iota) & (2 * B - 1)
        return 0

    lax.fori_loop(0, 2 * C // L, prefill, 0)

    def out_chunk(q, _):
        word0 = 2 * (lo + q * C)
        pltpu.sync_copy(mp.at[pl.ds(word0, 2 * C)], pos0)
        pltpu.sync_copy(mv.at[pl.ds(word0, 2 * C)], vel0)

        def fuse(k, _):
            for u in range(2):
                s = pl.ds(k * 2 * L + u * L, L)
                pos0[s] = pos0[s] + T * vel0[s]
            return 0

        lax.fori_loop(0, C // L, fuse, 0)

        def compact(k, cnt_splat):
            wv = w[pl.ds(q * C + k * L, L)]
            m = wv >= 0
            pc = plsc.all_reduce_population_count(m)

            @pl.when(pc[0] != 0)
            def _store():
                cum = plsc.cumsum(m.astype(jnp.int32))
                pos = cnt_splat + cum - 1
                plsc.store_scatter(gj, [pos], wv, mask=m)
                plsc.store_scatter(gp, [pos], k * L + iota, mask=m)

            return cnt_splat + pc

        mcnt_splat = lax.fori_loop(0, C // L, compact,
                                   jnp.zeros((L,), jnp.int32))
        mcnt = to_scalar(mcnt_splat)
        ng = (mcnt + (L - 1)) >> 4

        sub = iota >> 1
        colh = (iota & 1) * 128

        def expand(g, _):
            e0 = g * L
            jv16 = gj[pl.ds(e0, L)] & (B - 1)
            for s in range(2):
                jv8 = _dg(jv16, 8 * s + sub)
                wv_ = 256 * (jv8 >> 7) + (jv8 & 127) + colh
                gi[pl.ds(2 * e0 + L * s, L)] = wv_
            return 0

        lax.fori_loop(0, ng, expand, 0)

        nb = (2 * mcnt + (G - 1)) >> 9

        def gath(b, _):
            s = pl.ds(b * G, G)
            pltpu.async_copy(vp.at[gi.at[s]], gpx.at[s], gsem).wait()
            pltpu.async_copy(vv.at[gi.at[s]], gvx.at[s], gsem).wait()
            return 0

        lax.fori_loop(0, nb, gath, 0)

        def apply(g, _):
            e0 = g * L
            ev = e0 + iota
            prv = gp[pl.ds(e0, L)] & (C - 1)
            xs = plsc.load_gather(gpx, [2 * ev])
            ys = plsc.load_gather(gpx, [2 * ev + 1])
            vxs = plsc.load_gather(gvx, [2 * ev])
            vys = plsc.load_gather(gvx, [2 * ev + 1])
            ox = xs + T * vxs
            oy = ys + T * vys
            tgtx = 256 * (prv >> 7) + (prv & 127)
            m = ev < mcnt
            plsc.store_scatter(pos0, [tgtx], ox, mask=m)
            plsc.store_scatter(pos0, [tgtx + 128], oy, mask=m)
            return 0

        lax.fori_loop(0, ng, apply, 0)

        pltpu.sync_copy(pos0, out.at[pl.ds(word0, 2 * C)])
        return 0

    lax.fori_loop(0, NQ, out_chunk, 0)


@functools.partial(jax.jit, donate_argnums=())
def _run(mpf, mvf, vpf, vvf, idx):
    mesh = plsc.VectorSubcoreMesh(core_axis_name="c", subcore_axis_name="s")
    f = pl.kernel(
        _body,
        mesh=mesh,
        compiler_params=pltpu.CompilerParams(
            needs_layout_passes=False, use_tc_tiling_on_sc=False),
        out_type=jax.ShapeDtypeStruct((2 * N,), jnp.float32),
        scratch_types=[
            pltpu.VMEM((R,), jnp.int32),
            pltpu.VMEM((IC,), jnp.int32),
            pltpu.VMEM((IC + L,), jnp.int32),
            pltpu.VMEM((IC + L,), jnp.int32),
            pltpu.VMEM((2 * C,), jnp.float32),
            pltpu.VMEM((2 * C,), jnp.float32),
            pltpu.VMEM((C + L,), jnp.int32),
            pltpu.VMEM((C + L,), jnp.int32),
            pltpu.VMEM((2 * C,), jnp.int32),
            pltpu.VMEM((2 * C,), jnp.float32),
            pltpu.VMEM((2 * C,), jnp.float32),
            pltpu.SemaphoreType.DMA,
        ],
    )
    return f(mpf, mvf, vpf, vvf, idx)


def _tile_order_flat(a):
    n = a.shape[0]
    return a.reshape(n // 128, 128, 2).transpose(0, 2, 1).reshape(-1)


def kernel(mem_pos, mem_vel, val_pos, val_vel, idx):
    out = _run(_tile_order_flat(mem_pos), _tile_order_flat(mem_vel),
               _tile_order_flat(val_pos), _tile_order_flat(val_vel), idx)
    return out.reshape(N // 128, 2, 128).transpose(0, 2, 1).reshape(N, 2)


# speedup vs baseline: 1.4517x; 1.4517x over previous
import functools

import jax
import jax.numpy as jnp
from jax import lax
from jax.experimental import pallas as pl
from jax.experimental.pallas import tpu as pltpu
from jax.experimental.pallas import tpu_sc as plsc

N = 1048576
B = 131072
T = 0.5

NC = 2
NS = 16
NW = NC * NS
R = N // NW
IC = 8192
NCH = B // IC
C = 4096
NQ = R // C
L = 16
G = 512


def _dg(a, i):
    dnums = lax.GatherDimensionNumbers(
        offset_dims=(), collapsed_slice_dims=(0,), start_index_map=(0,))
    return lax.gather(a, i[:, None], dnums, (1,),
                      mode=lax.GatherScatterMode.PROMISE_IN_BOUNDS)


def _body(mp, mv, vp, vv, ix, out, w, idx0, civ, cjv, pos0, vel0, gj, gp, gi,
          gpx, gvx, gsem):
    wid = lax.axis_index("s") * NC + lax.axis_index("c")
    lo = wid * R
    iota = lax.iota(jnp.int32, L)
    lane15 = jnp.full((L,), L - 1, jnp.int32)

    def to_scalar(splat):
        return splat[L - 1]

    neg1 = jnp.full((L,), -1, jnp.int32)

    def init_w(i, _):
        for u in range(4):
            w[pl.ds(i * 4 * L + u * L, L)] = neg1
        return 0

    lax.fori_loop(0, R // (4 * L), init_w, 0)

    def scan_chunk(c, _):
        pltpu.sync_copy(ix.at[pl.ds(c * IC, IC)], idx0)

        def scan_vreg(k, cnt_splat):
            iv = idx0[pl.ds(k * L, L)]
            jv = c * IC + k * L + iota
            m = (iv >= lo) & (iv < lo + R)
            pc = plsc.all_reduce_population_count(m)
            cum = plsc.cumsum(m.astype(jnp.int32))
            pos = cnt_splat + cum - 1
            plsc.store_scatter(civ, [pos], iv - lo, mask=m)
            plsc.store_scatter(cjv, [pos], jv, mask=m)
            return cnt_splat + pc

        cnt_splat = lax.fori_loop(0, IC // L, scan_vreg,
                                  jnp.zeros((L,), jnp.int32))
        cnt = to_scalar(cnt_splat)

        def apply_vreg(g, _):
            base_l = g * L
            av = civ[pl.ds(base_l, L)] & (R - 1)
            bv = cjv[pl.ds(base_l, L)]
            rem = cnt - base_l
            lm = iota < rem
            loser = jnp.zeros((L,), jnp.bool_)
            for r in range(1, L):
                rot = _dg(av, (iota + r) & (L - 1))
                eq = rot == av
                ok = (iota < (L - r)) & (iota < (rem - r))
                loser = loser | (eq & ok)
            win = lm & jnp.logical_not(loser)
            plsc.store_scatter(w, [av], bv, mask=win)
            return 0

        ng = (cnt + (L - 1)) >> 4
        lax.fori_loop(0, ng, apply_vreg, 0)
        return 0

    lax.fori_loop(0, NCH, scan_chunk, 0)

    def prefill(k, _):
        gi[pl.ds(k * L, L)] = (wid * 2 * C + k * L + iota) & (2 * B - 1)
        return 0

    lax.fori_loop(0, 2 * C // L, prefill, 0)

    def out_chunk(q, _):
        word0 = 2 * (lo + q * C)
        pltpu.sync_copy(mp.at[pl.ds(word0, 2 * C)], pos0)
        pltpu.sync_copy(mv.at[pl.ds(word0, 2 * C)], vel0)

        def fuse(k, _):
            for u in range(2):
                s = pl.ds(k * 2 * L + u * L, L)
                pos0[s] = pos0[s] + T * vel0[s]
            return 0

        lax.fori_loop(0, C // L, fuse, 0)

        def compact(k, cnt_splat):
            wv = w[pl.ds(q * C + k * L, L)]
            m = wv >= 0
            pc = plsc.all_reduce_population_count(m)
            cum = plsc.cumsum(m.astype(jnp.int32))
            pos = cnt_splat + cum - 1
            plsc.store_scatter(gj, [pos], wv, mask=m)
            plsc.store_scatter(gp, [pos], k * L + iota, mask=m)
            return cnt_splat + pc

        mcnt_splat = lax.fori_loop(0, C // L, compact,
                                   jnp.zeros((L,), jnp.int32))
        mcnt = to_scalar(mcnt_splat)
        ng = (mcnt + (L - 1)) >> 4

        sub = iota >> 1
        colh = (iota & 1) * 128

        def expand(g, _):
            e0 = g * L
            jv16 = gj[pl.ds(e0, L)] & (B - 1)
            for s in range(2):
                jv8 = _dg(jv16, 8 * s + sub)
                wv_ = 256 * (jv8 >> 7) + (jv8 & 127) + colh
                gi[pl.ds(2 * e0 + L * s, L)] = wv_
            return 0

        lax.fori_loop(0, ng, expand, 0)

        nb = (2 * mcnt + (G - 1)) >> 9

        def gath(b, _):
            s = pl.ds(b * G, G)
            pltpu.async_copy(vp.at[gi.at[s]], gpx.at[s], gsem).wait()
            pltpu.async_copy(vv.at[gi.at[s]], gvx.at[s], gsem).wait()
            return 0

        lax.fori_loop(0, nb, gath, 0)

        def apply(g, _):
            e0 = g * L
            ev = e0 + iota
            prv = gp[pl.ds(e0, L)] & (C - 1)
            xs = plsc.load_gather(gpx, [2 * ev])
            ys = plsc.load_gather(gpx, [2 * ev + 1])
            vxs = plsc.load_gather(gvx, [2 * ev])
            vys = plsc.load_gather(gvx, [2 * ev + 1])
            ox = xs + T * vxs
            oy = ys + T * vys
            tgtx = 256 * (prv >> 7) + (prv & 127)
            m = ev < mcnt
            plsc.store_scatter(pos0, [tgtx], ox, mask=m)
            plsc.store_scatter(pos0, [tgtx + 128], oy, mask=m)
            return 0

        lax.fori_loop(0, ng, apply, 0)

        pltpu.sync_copy(pos0, out.at[pl.ds(word0, 2 * C)])
        return 0

    lax.fori_loop(0, NQ, out_chunk, 0)


@functools.partial(jax.jit, donate_argnums=())
def _run(mpf, mvf, vpf, vvf, idx):
    mesh = plsc.VectorSubcoreMesh(core_axis_name="c", subcore_axis_name="s")
    f = pl.kernel(
        _body,
        mesh=mesh,
        compiler_params=pltpu.CompilerParams(
            needs_layout_passes=False, use_tc_tiling_on_sc=False),
        out_type=jax.ShapeDtypeStruct((2 * N,), jnp.float32),
        scratch_types=[
            pltpu.VMEM((R,), jnp.int32),
            pltpu.VMEM((IC,), jnp.int32),
            pltpu.VMEM((IC + L,), jnp.int32),
            pltpu.VMEM((IC + L,), jnp.int32),
            pltpu.VMEM((2 * C,), jnp.float32),
            pltpu.VMEM((2 * C,), jnp.float32),
            pltpu.VMEM((C + L,), jnp.int32),
            pltpu.VMEM((C + L,), jnp.int32),
            pltpu.VMEM((2 * C,), jnp.int32),
            pltpu.VMEM((2 * C,), jnp.float32),
            pltpu.VMEM((2 * C,), jnp.float32),
            pltpu.SemaphoreType.DMA,
        ],
    )
    return f(mpf, mvf, vpf, vvf, idx)


def _tile_order_flat(a):
    n = a.shape[0]
    return a.reshape(n // 128, 128, 2).transpose(0, 2, 1).reshape(-1)


def kernel(mem_pos, mem_vel, val_pos, val_vel, idx):
    out = _run(_tile_order_flat(mem_pos), _tile_order_flat(mem_vel),
               _tile_order_flat(val_pos), _tile_order_flat(val_vel), idx)
    return out.reshape(N // 128, 2, 128).transpose(0, 2, 1).reshape(N, 2)


# speedup vs baseline: 1.6515x; 1.1377x over previous
import functools

import jax
import jax.numpy as jnp
from jax import lax
from jax.experimental import pallas as pl
from jax.experimental.pallas import tpu as pltpu
from jax.experimental.pallas import tpu_sc as plsc

N = 1048576
B = 131072
T = 0.5

NC = 2
NS = 16
NW = NC * NS
R = N // NW
IC = 8192
NCH = B // IC
C = 4096
NQ = R // C
L = 16
G = 512


def _dg(a, i):
    dnums = lax.GatherDimensionNumbers(
        offset_dims=(), collapsed_slice_dims=(0,), start_index_map=(0,))
    return lax.gather(a, i[:, None], dnums, (1,),
                      mode=lax.GatherScatterMode.PROMISE_IN_BOUNDS)


def _body(mp, mv, vp, vv, ix, out, w, idx0, civ, cjv, pos0, vel0, gj, gp, gi,
          gpx, gvx, sema, semb):
    wid = lax.axis_index("s") * NC + lax.axis_index("c")
    lo = wid * R
    iota = lax.iota(jnp.int32, L)
    lane15 = jnp.full((L,), L - 1, jnp.int32)

    def to_scalar(splat):
        return splat[L - 1]

    neg1 = jnp.full((L,), -1, jnp.int32)

    def init_w(i, _):
        for u in range(4):
            w[pl.ds(i * 4 * L + u * L, L)] = neg1
        return 0

    lax.fori_loop(0, R // (4 * L), init_w, 0)

    def ix_copy(c, buf, sem):
        cc = jnp.minimum(c, NCH - 1)
        return pltpu.make_async_copy(ix.at[pl.ds(cc * IC, IC)], buf, sem)

    ix_copy(0, idx0.at[0], sema).start()
    ix_copy(1, idx0.at[1], semb).start()

    def scan_chunk(c, par):
        buf, sem = par

        def scan_vreg(k, cnt_splat):
            iv = buf[pl.ds(k * L, L)]
            jv = c * IC + k * L + iota
            m = (iv >= lo) & (iv < lo + R)
            pc = plsc.all_reduce_population_count(m)
            cum = plsc.cumsum(m.astype(jnp.int32))
            pos = cnt_splat + cum - 1
            plsc.store_scatter(civ, [pos], iv - lo, mask=m)
            plsc.store_scatter(cjv, [pos], jv, mask=m)
            return cnt_splat + pc

        ix_copy(c, buf, sem).wait()
        cnt_splat = lax.fori_loop(0, IC // L, scan_vreg,
                                  jnp.zeros((L,), jnp.int32))
        ix_copy(c + 2, buf, sem).start()
        cnt = to_scalar(cnt_splat)

        def apply_vreg(g, _):
            base_l = g * L
            av = civ[pl.ds(base_l, L)] & (R - 1)
            bv = cjv[pl.ds(base_l, L)]
            rem = cnt - base_l
            lm = iota < rem
            loser = jnp.zeros((L,), jnp.bool_)
            for r in range(1, L):
                rot = _dg(av, (iota + r) & (L - 1))
                eq = rot == av
                ok = (iota < (L - r)) & (iota < (rem - r))
                loser = loser | (eq & ok)
            win = lm & jnp.logical_not(loser)
            plsc.store_scatter(w, [av], bv, mask=win)
            return 0

        ng = (cnt + (L - 1)) >> 4
        lax.fori_loop(0, ng, apply_vreg, 0)
        return 0

    for c in range(NCH):
        scan_chunk(c, (idx0.at[c % 2], sema if c % 2 == 0 else semb))
    ix_copy(NCH, idx0.at[0], sema).wait()
    ix_copy(NCH + 1, idx0.at[1], semb).wait()

    def prefill(k, _):
        gi[pl.ds(k * L, L)] = (wid * 2 * C + k * L + iota) & (2 * B - 1)
        return 0

    lax.fori_loop(0, 2 * C // L, prefill, 0)

    def out_chunk(q, _):
        word0 = 2 * (lo + q * C)
        dp = pltpu.make_async_copy(mp.at[pl.ds(word0, 2 * C)], pos0, sema)
        dv = pltpu.make_async_copy(mv.at[pl.ds(word0, 2 * C)], vel0, semb)
        dp.start()
        dv.start()
        dp.wait()
        dv.wait()

        def fuse(k, _):
            for u in range(2):
                s = pl.ds(k * 2 * L + u * L, L)
                pos0[s] = pos0[s] + T * vel0[s]
            return 0

        lax.fori_loop(0, C // L, fuse, 0)

        def compact(k, cnt_splat):
            wv = w[pl.ds(q * C + k * L, L)]
            m = wv >= 0
            pc = plsc.all_reduce_population_count(m)
            cum = plsc.cumsum(m.astype(jnp.int32))
            pos = cnt_splat + cum - 1
            plsc.store_scatter(gj, [pos], wv, mask=m)
            plsc.store_scatter(gp, [pos], k * L + iota, mask=m)
            return cnt_splat + pc

        mcnt_splat = lax.fori_loop(0, C // L, compact,
                                   jnp.zeros((L,), jnp.int32))
        mcnt = to_scalar(mcnt_splat)
        ng = (mcnt + (L - 1)) >> 4

        sub = iota >> 1
        colh = (iota & 1) * 128

        def expand(g, _):
            e0 = g * L
            jv16 = gj[pl.ds(e0, L)] & (B - 1)
            for s in range(2):
                jv8 = _dg(jv16, 8 * s + sub)
                wv_ = 256 * (jv8 >> 7) + (jv8 & 127) + colh
                gi[pl.ds(2 * e0 + L * s, L)] = wv_
            return 0

        lax.fori_loop(0, ng, expand, 0)

        nb = (2 * mcnt + (G - 1)) >> 9

        def gath(b, _):
            s = pl.ds(b * G, G)
            d1 = pltpu.make_async_copy(vp.at[gi.at[s]], gpx.at[s], sema)
            d2 = pltpu.make_async_copy(vv.at[gi.at[s]], gvx.at[s], semb)
            d1.start()
            d2.start()
            d1.wait()
            d2.wait()
            return 0

        lax.fori_loop(0, nb, gath, 0)

        def apply(g, _):
            e0 = g * L
            ev = e0 + iota
            prv = gp[pl.ds(e0, L)] & (C - 1)
            xs = plsc.load_gather(gpx, [2 * ev])
            ys = plsc.load_gather(gpx, [2 * ev + 1])
            vxs = plsc.load_gather(gvx, [2 * ev])
            vys = plsc.load_gather(gvx, [2 * ev + 1])
            ox = xs + T * vxs
            oy = ys + T * vys
            tgtx = 256 * (prv >> 7) + (prv & 127)
            m = ev < mcnt
            plsc.store_scatter(pos0, [tgtx], ox, mask=m)
            plsc.store_scatter(pos0, [tgtx + 128], oy, mask=m)
            return 0

        lax.fori_loop(0, ng, apply, 0)

        pltpu.sync_copy(pos0, out.at[pl.ds(word0, 2 * C)])
        return 0

    lax.fori_loop(0, NQ, out_chunk, 0)


@functools.partial(jax.jit, donate_argnums=())
def _run(mpf, mvf, vpf, vvf, idx):
    mesh = plsc.VectorSubcoreMesh(core_axis_name="c", subcore_axis_name="s")
    f = pl.kernel(
        _body,
        mesh=mesh,
        compiler_params=pltpu.CompilerParams(
            needs_layout_passes=False, use_tc_tiling_on_sc=False),
        out_type=jax.ShapeDtypeStruct((2 * N,), jnp.float32),
        scratch_types=[
            pltpu.VMEM((R,), jnp.int32),
            pltpu.VMEM((2, IC), jnp.int32),
            pltpu.VMEM((IC + L,), jnp.int32),
            pltpu.VMEM((IC + L,), jnp.int32),
            pltpu.VMEM((2 * C,), jnp.float32),
            pltpu.VMEM((2 * C,), jnp.float32),
            pltpu.VMEM((C + L,), jnp.int32),
            pltpu.VMEM((C + L,), jnp.int32),
            pltpu.VMEM((2 * C,), jnp.int32),
            pltpu.VMEM((2 * C,), jnp.float32),
            pltpu.VMEM((2 * C,), jnp.float32),
            pltpu.SemaphoreType.DMA,
            pltpu.SemaphoreType.DMA,
        ],
    )
    return f(mpf, mvf, vpf, vvf, idx)


def _tile_order_flat(a):
    n = a.shape[0]
    return a.reshape(n // 128, 128, 2).transpose(0, 2, 1).reshape(-1)


def kernel(mem_pos, mem_vel, val_pos, val_vel, idx):
    out = _run(_tile_order_flat(mem_pos), _tile_order_flat(mem_vel),
               _tile_order_flat(val_pos), _tile_order_flat(val_vel), idx)
    return out.reshape(N // 128, 2, 128).transpose(0, 2, 1).reshape(N, 2)


# speedup vs baseline: 1.6694x; 1.0109x over previous
import functools

import jax
import jax.numpy as jnp
from jax import lax
from jax.experimental import pallas as pl
from jax.experimental.pallas import tpu as pltpu
from jax.experimental.pallas import tpu_sc as plsc

N = 1048576
B = 131072
T = 0.5

NC = 2
NS = 16
NW = NC * NS
R = N // NW
IC = 8192
NCH = B // IC
C = 2048
NQ = R // C
L = 16
G = 512


def _dg(a, i):
    dnums = lax.GatherDimensionNumbers(
        offset_dims=(), collapsed_slice_dims=(0,), start_index_map=(0,))
    return lax.gather(a, i[:, None], dnums, (1,),
                      mode=lax.GatherScatterMode.PROMISE_IN_BOUNDS)


def _body(mp, mv, vp, vv, ix, out, w, idx0, civ, cjv, pos0, vel0, gj, gp, gi,
          gpx, gvx, sema, semb, sp0, sp1, sv0, sv1, so0, so1):
    sem_p = (sp0, sp1)
    sem_v = (sv0, sv1)
    sem_o = (so0, so1)
    sem_g1 = sema
    sem_g2 = semb
    wid = lax.axis_index("s") * NC + lax.axis_index("c")
    lo = wid * R
    iota = lax.iota(jnp.int32, L)
    lane15 = jnp.full((L,), L - 1, jnp.int32)

    def to_scalar(splat):
        return splat[L - 1]

    neg1 = jnp.full((L,), -1, jnp.int32)

    def init_w(i, _):
        for u in range(4):
            w[pl.ds(i * 4 * L + u * L, L)] = neg1
        return 0

    lax.fori_loop(0, R // (4 * L), init_w, 0)

    def ix_copy(c, buf, sem):
        cc = jnp.minimum(c, NCH - 1)
        return pltpu.make_async_copy(ix.at[pl.ds(cc * IC, IC)], buf, sem)

    ix_copy(0, idx0.at[0], sema).start()
    ix_copy(1, idx0.at[1], semb).start()

    def scan_chunk(c, par):
        buf, sem = par

        def scan_vreg(k, cnt_splat):
            iv = buf[pl.ds(k * L, L)]
            jv = c * IC + k * L + iota
            m = (iv >= lo) & (iv < lo + R)
            pc = plsc.all_reduce_population_count(m)
            cum = plsc.cumsum(m.astype(jnp.int32))
            pos = cnt_splat + cum - 1
            plsc.store_scatter(civ, [pos], iv - lo, mask=m)
            plsc.store_scatter(cjv, [pos], jv, mask=m)
            return cnt_splat + pc

        ix_copy(c, buf, sem).wait()
        cnt_splat = lax.fori_loop(0, IC // L, scan_vreg,
                                  jnp.zeros((L,), jnp.int32))
        ix_copy(c + 2, buf, sem).start()
        cnt = to_scalar(cnt_splat)

        def apply_vreg(g, _):
            base_l = g * L
            av = civ[pl.ds(base_l, L)] & (R - 1)
            bv = cjv[pl.ds(base_l, L)]
            rem = cnt - base_l
            lm = iota < rem
            loser = jnp.zeros((L,), jnp.bool_)
            for r in range(1, L):
                rot = _dg(av, (iota + r) & (L - 1))
                eq = rot == av
                ok = (iota < (L - r)) & (iota < (rem - r))
                loser = loser | (eq & ok)
            win = lm & jnp.logical_not(loser)
            plsc.store_scatter(w, [av], bv, mask=win)
            return 0

        ng = (cnt + (L - 1)) >> 4
        lax.fori_loop(0, ng, apply_vreg, 0)
        return 0

    for c in range(NCH):
        scan_chunk(c, (idx0.at[c % 2], sema if c % 2 == 0 else semb))
    ix_copy(NCH, idx0.at[0], sema).wait()
    ix_copy(NCH + 1, idx0.at[1], semb).wait()

    def prefill(k, _):
        gi[pl.ds(k * L, L)] = (wid * 2 * C + k * L + iota) & (2 * B - 1)
        return 0

    lax.fori_loop(0, 2 * C // L, prefill, 0)

    def ld(q, b):
        qq = min(q, NQ - 1)
        w0 = 2 * (lo + qq * C)
        return (pltpu.make_async_copy(mp.at[pl.ds(w0, 2 * C)], pos0.at[b], sem_p[b]),
                pltpu.make_async_copy(mv.at[pl.ds(w0, 2 * C)], vel0.at[b], sem_v[b]))

    def st(q, b):
        w0 = 2 * (lo + q * C)
        return pltpu.make_async_copy(pos0.at[b], out.at[pl.ds(w0, 2 * C)],
                                     sem_o[b])

    for d in ld(0, 0):
        d.start()

    for q in range(NQ):
        b = q & 1
        pb = pos0.at[b]
        vb = vel0.at[b]
        for d in ld(q, b):
            d.wait()
        if q + 1 < NQ:
            if q >= 1:
                st(q - 1, 1 - b).wait()
            for d in ld(q + 1, 1 - b):
                d.start()
        def fuse(k, _, pb=pb, vb=vb):
            for u in range(2):
                s = pl.ds(k * 2 * L + u * L, L)
                pb[s] = pb[s] + T * vb[s]
            return 0

        lax.fori_loop(0, C // L, fuse, 0)

        def compact(k, cnt_splat, q=q):
            wv = w[pl.ds(q * C + k * L, L)]
            m = wv >= 0
            pc = plsc.all_reduce_population_count(m)
            cum = plsc.cumsum(m.astype(jnp.int32))
            pos = cnt_splat + cum - 1
            plsc.store_scatter(gj, [pos], wv, mask=m)
            plsc.store_scatter(gp, [pos], k * L + iota, mask=m)
            return cnt_splat + pc

        mcnt_splat = lax.fori_loop(0, C // L, compact,
                                   jnp.zeros((L,), jnp.int32))
        mcnt = to_scalar(mcnt_splat)
        ng = (mcnt + (L - 1)) >> 4

        sub = iota >> 1
        colh = (iota & 1) * 128

        def expand(g, _):
            e0 = g * L
            jv16 = gj[pl.ds(e0, L)] & (B - 1)
            for s in range(2):
                jv8 = _dg(jv16, 8 * s + sub)
                wv_ = 256 * (jv8 >> 7) + (jv8 & 127) + colh
                gi[pl.ds(2 * e0 + L * s, L)] = wv_
            return 0

        lax.fori_loop(0, ng, expand, 0)

        nb = (2 * mcnt + (G - 1)) >> 9

        def gath(bb, _):
            s = pl.ds(bb * G, G)
            d1 = pltpu.make_async_copy(vp.at[gi.at[s]], gpx.at[s], sem_g1)
            d2 = pltpu.make_async_copy(vv.at[gi.at[s]], gvx.at[s], sem_g2)
            d1.start()
            d2.start()
            d1.wait()
            d2.wait()
            return 0

        lax.fori_loop(0, nb, gath, 0)

        def apply(g, _, pb=pb, mcnt=mcnt):
            e0 = g * L
            ev = e0 + iota
            prv = gp[pl.ds(e0, L)] & (C - 1)
            xs = plsc.load_gather(gpx, [2 * ev])
            ys = plsc.load_gather(gpx, [2 * ev + 1])
            vxs = plsc.load_gather(gvx, [2 * ev])
            vys = plsc.load_gather(gvx, [2 * ev + 1])
            ox = xs + T * vxs
            oy = ys + T * vys
            tgtx = 256 * (prv >> 7) + (prv & 127)
            m = ev < mcnt
            plsc.store_scatter(pb, [tgtx], ox, mask=m)
            plsc.store_scatter(pb, [tgtx + 128], oy, mask=m)
            return 0

        lax.fori_loop(0, ng, apply, 0)

        st(q, b).start()

    st(NQ - 1, (NQ - 1) & 1).wait()
    st(NQ - 2, (NQ - 2) & 1).wait()


@functools.partial(jax.jit, donate_argnums=())
def _run(mpf, mvf, vpf, vvf, idx):
    mesh = plsc.VectorSubcoreMesh(core_axis_name="c", subcore_axis_name="s")
    f = pl.kernel(
        _body,
        mesh=mesh,
        compiler_params=pltpu.CompilerParams(
            needs_layout_passes=False, use_tc_tiling_on_sc=False),
        out_type=jax.ShapeDtypeStruct((2 * N,), jnp.float32),
        scratch_types=[
            pltpu.VMEM((R,), jnp.int32),
            pltpu.VMEM((2, IC), jnp.int32),
            pltpu.VMEM((IC + L,), jnp.int32),
            pltpu.VMEM((IC + L,), jnp.int32),
            pltpu.VMEM((2, 2 * C), jnp.float32),
            pltpu.VMEM((2, 2 * C), jnp.float32),
            pltpu.VMEM((C + L,), jnp.int32),
            pltpu.VMEM((C + L,), jnp.int32),
            pltpu.VMEM((2 * C,), jnp.int32),
            pltpu.VMEM((2 * C,), jnp.float32),
            pltpu.VMEM((2 * C,), jnp.float32),
            pltpu.SemaphoreType.DMA,
            pltpu.SemaphoreType.DMA,
            pltpu.SemaphoreType.DMA,
            pltpu.SemaphoreType.DMA,
            pltpu.SemaphoreType.DMA,
            pltpu.SemaphoreType.DMA,
            pltpu.SemaphoreType.DMA,
            pltpu.SemaphoreType.DMA,
        ],
    )
    return f(mpf, mvf, vpf, vvf, idx)


def _tile_order_flat(a):
    n = a.shape[0]
    return a.reshape(n // 128, 128, 2).transpose(0, 2, 1).reshape(-1)


def kernel(mem_pos, mem_vel, val_pos, val_vel, idx):
    out = _run(_tile_order_flat(mem_pos), _tile_order_flat(mem_vel),
               _tile_order_flat(val_pos), _tile_order_flat(val_vel), idx)
    return out.reshape(N // 128, 2, 128).transpose(0, 2, 1).reshape(N, 2)


# speedup vs baseline: 1.6705x; 1.0006x over previous
import functools

import jax
import jax.numpy as jnp
from jax import lax
from jax.experimental import pallas as pl
from jax.experimental.pallas import tpu as pltpu
from jax.experimental.pallas import tpu_sc as plsc

N = 1048576
B = 131072
T = 0.5

NC = 2
NS = 16
NW = NC * NS
R = N // NW
IC = 8192
NCH = B // IC
C = 2048
NQ = R // C
L = 16
G = 512


def _dg(a, i):
    dnums = lax.GatherDimensionNumbers(
        offset_dims=(), collapsed_slice_dims=(0,), start_index_map=(0,))
    return lax.gather(a, i[:, None], dnums, (1,),
                      mode=lax.GatherScatterMode.PROMISE_IN_BOUNDS)


def _body(mp, mv, vp, vv, ix, out, w, idx0, civ, cjv, pos0, vel0, gj, gp, gi,
          gpx, gvx, sema, semb, sp0, sp1, sv0, sv1, so0, so1):
    sem_p = (sp0, sp1)
    sem_v = (sv0, sv1)
    sem_o = (so0, so1)
    sem_g1 = sema
    sem_g2 = semb
    wid = lax.axis_index("s") * NC + lax.axis_index("c")
    lo = wid * R
    iota = lax.iota(jnp.int32, L)
    lane15 = jnp.full((L,), L - 1, jnp.int32)

    def to_scalar(splat):
        return splat[L - 1]

    neg1 = jnp.full((L,), -1, jnp.int32)

    def init_w(i, _):
        for u in range(4):
            w[pl.ds(i * 4 * L + u * L, L)] = neg1
        return 0

    lax.fori_loop(0, R // (4 * L), init_w, 0)

    def ix_copy(c, buf, sem):
        cc = jnp.minimum(c, NCH - 1)
        return pltpu.make_async_copy(ix.at[pl.ds(cc * IC, IC)], buf, sem)

    ix_copy(0, idx0.at[0], sema).start()
    ix_copy(1, idx0.at[1], semb).start()

    def scan_chunk(c, par):
        buf, sem = par

        def scan_vreg(k, cnt_splat):
            for u in range(2):
                iv = buf[pl.ds(k * 2 * L + u * L, L)]
                jv = c * IC + k * 2 * L + u * L + iota
                m = (iv >= lo) & (iv < lo + R)
                pc = plsc.all_reduce_population_count(m)
                cum = plsc.cumsum(m.astype(jnp.int32))
                pos = cnt_splat + cum - 1
                plsc.store_scatter(civ, [pos], iv - lo, mask=m)
                plsc.store_scatter(cjv, [pos], jv, mask=m)
                cnt_splat = cnt_splat + pc
            return cnt_splat

        ix_copy(c, buf, sem).wait()
        cnt_splat = lax.fori_loop(0, IC // (2 * L), scan_vreg,
                                  jnp.zeros((L,), jnp.int32))
        ix_copy(c + 2, buf, sem).start()
        cnt = to_scalar(cnt_splat)

        def apply_vreg(g, _):
            base_l = g * L
            av = civ[pl.ds(base_l, L)] & (R - 1)
            bv = cjv[pl.ds(base_l, L)]
            rem = cnt - base_l
            lm = iota < rem
            loser = jnp.zeros((L,), jnp.bool_)
            for r in range(1, L):
                rot = _dg(av, (iota + r) & (L - 1))
                eq = rot == av
                ok = (iota < (L - r)) & (iota < (rem - r))
                loser = loser | (eq & ok)
            win = lm & jnp.logical_not(loser)
            plsc.store_scatter(w, [av], bv, mask=win)
            return 0

        ng = (cnt + (L - 1)) >> 4
        lax.fori_loop(0, ng, apply_vreg, 0)
        return 0

    for c in range(NCH):
        scan_chunk(c, (idx0.at[c % 2], sema if c % 2 == 0 else semb))
    ix_copy(NCH, idx0.at[0], sema).wait()
    ix_copy(NCH + 1, idx0.at[1], semb).wait()

    def prefill(k, _):
        gi[pl.ds(k * L, L)] = (wid * 2 * C + k * L + iota) & (2 * B - 1)
        return 0

    lax.fori_loop(0, 2 * C // L, prefill, 0)

    def ld(q, b):
        qq = min(q, NQ - 1)
        w0 = 2 * (lo + qq * C)
        return (pltpu.make_async_copy(mp.at[pl.ds(w0, 2 * C)], pos0.at[b], sem_p[b]),
                pltpu.make_async_copy(mv.at[pl.ds(w0, 2 * C)], vel0.at[b], sem_v[b]))

    def st(q, b):
        w0 = 2 * (lo + q * C)
        return pltpu.make_async_copy(pos0.at[b], out.at[pl.ds(w0, 2 * C)],
                                     sem_o[b])

    for d in ld(0, 0):
        d.start()

    for q in range(NQ):
        b = q & 1
        pb = pos0.at[b]
        vb = vel0.at[b]
        for d in ld(q, b):
            d.wait()
        if q + 1 < NQ:
            if q >= 1:
                st(q - 1, 1 - b).wait()
            for d in ld(q + 1, 1 - b):
                d.start()
        def fuse(k, _, pb=pb, vb=vb):
            for u in range(2):
                s = pl.ds(k * 2 * L + u * L, L)
                pb[s] = pb[s] + T * vb[s]
            return 0

        lax.fori_loop(0, C // L, fuse, 0)

        def compact(k, cnt_splat, q=q):
            wv = w[pl.ds(q * C + k * L, L)]
            m = wv >= 0
            pc = plsc.all_reduce_population_count(m)
            cum = plsc.cumsum(m.astype(jnp.int32))
            pos = cnt_splat + cum - 1
            plsc.store_scatter(gj, [pos], wv, mask=m)
            plsc.store_scatter(gp, [pos], k * L + iota, mask=m)
            return cnt_splat + pc

        mcnt_splat = lax.fori_loop(0, C // L, compact,
                                   jnp.zeros((L,), jnp.int32))
        mcnt = to_scalar(mcnt_splat)
        ng = (mcnt + (L - 1)) >> 4

        sub = iota >> 1
        colh = (iota & 1) * 128

        def expand(g, _):
            e0 = g * L
            jv16 = gj[pl.ds(e0, L)] & (B - 1)
            for s in range(2):
                jv8 = _dg(jv16, 8 * s + sub)
                wv_ = 256 * (jv8 >> 7) + (jv8 & 127) + colh
                gi[pl.ds(2 * e0 + L * s, L)] = wv_
            return 0

        lax.fori_loop(0, ng, expand, 0)

        nb = (2 * mcnt + (G - 1)) >> 9

        def gath(bb, _):
            s = pl.ds(bb * G, G)
            d1 = pltpu.make_async_copy(vp.at[gi.at[s]], gpx.at[s], sem_g1)
            d2 = pltpu.make_async_copy(vv.at[gi.at[s]], gvx.at[s], sem_g2)
            d1.start()
            d2.start()
            d1.wait()
            d2.wait()
            return 0

        lax.fori_loop(0, nb, gath, 0)

        def apply(g, _, pb=pb, mcnt=mcnt):
            e0 = g * L
            ev = e0 + iota
            prv = gp[pl.ds(e0, L)] & (C - 1)
            xs = plsc.load_gather(gpx, [2 * ev])
            ys = plsc.load_gather(gpx, [2 * ev + 1])
            vxs = plsc.load_gather(gvx, [2 * ev])
            vys = plsc.load_gather(gvx, [2 * ev + 1])
            ox = xs + T * vxs
            oy = ys + T * vys
            tgtx = 256 * (prv >> 7) + (prv & 127)
            m = ev < mcnt
            plsc.store_scatter(pb, [tgtx], ox, mask=m)
            plsc.store_scatter(pb, [tgtx + 128], oy, mask=m)
            return 0

        lax.fori_loop(0, ng, apply, 0)

        st(q, b).start()

    st(NQ - 1, (NQ - 1) & 1).wait()
    st(NQ - 2, (NQ - 2) & 1).wait()


@functools.partial(jax.jit, donate_argnums=())
def _run(mpf, mvf, vpf, vvf, idx):
    mesh = plsc.VectorSubcoreMesh(core_axis_name="c", subcore_axis_name="s")
    f = pl.kernel(
        _body,
        mesh=mesh,
        compiler_params=pltpu.CompilerParams(
            needs_layout_passes=False, use_tc_tiling_on_sc=False),
        out_type=jax.ShapeDtypeStruct((2 * N,), jnp.float32),
        scratch_types=[
            pltpu.VMEM((R,), jnp.int32),
            pltpu.VMEM((2, IC), jnp.int32),
            pltpu.VMEM((IC + L,), jnp.int32),
            pltpu.VMEM((IC + L,), jnp.int32),
            pltpu.VMEM((2, 2 * C), jnp.float32),
            pltpu.VMEM((2, 2 * C), jnp.float32),
            pltpu.VMEM((C + L,), jnp.int32),
            pltpu.VMEM((C + L,), jnp.int32),
            pltpu.VMEM((2 * C,), jnp.int32),
            pltpu.VMEM((2 * C,), jnp.float32),
            pltpu.VMEM((2 * C,), jnp.float32),
            pltpu.SemaphoreType.DMA,
            pltpu.SemaphoreType.DMA,
            pltpu.SemaphoreType.DMA,
            pltpu.SemaphoreType.DMA,
            pltpu.SemaphoreType.DMA,
            pltpu.SemaphoreType.DMA,
            pltpu.SemaphoreType.DMA,
            pltpu.SemaphoreType.DMA,
        ],
    )
    return f(mpf, mvf, vpf, vvf, idx)


def _tile_order_flat(a):
    n = a.shape[0]
    return a.reshape(n // 128, 128, 2).transpose(0, 2, 1).reshape(-1)


def kernel(mem_pos, mem_vel, val_pos, val_vel, idx):
    out = _run(_tile_order_flat(mem_pos), _tile_order_flat(mem_vel),
               _tile_order_flat(val_pos), _tile_order_flat(val_vel), idx)
    return out.reshape(N // 128, 2, 128).transpose(0, 2, 1).reshape(N, 2)
